# baseline (speedup 1.0000x reference)
import functools

import jax
import jax.numpy as jnp
from jax import lax
from jax.experimental import pallas as pl
from jax.experimental.pallas import tpu as pltpu
from jax.experimental.pallas import tpu_sc as plsc

N_PSI = 24
N_CONES = 192
N_SO3 = N_CONES * N_PSI
BLOCK_B = 2048


def _cone_kernel(pred_ref, grid_ref, out_ref):
    bb = pred_ref.shape[2]
    p9 = jnp.concatenate([pred_ref[0], pred_ref[1], pred_ref[2]], axis=0)
    g9 = jnp.concatenate([grid_ref[0], grid_ref[1], grid_ref[2]], axis=0)
    simT = jax.lax.dot_general(
        g9, p9, (((0,), (0,)), ((), ())),
        preferred_element_type=jnp.float32)
    m8 = jnp.max(simT.reshape(N_CONES, 3, 8, bb), axis=1)
    gmax = jnp.max(m8, axis=(0, 1), keepdims=True)
    cidx = jax.lax.broadcasted_iota(jnp.int32, (N_CONES, 1, 1), 0)
    cone = jnp.min(jnp.where(m8 == gmax, cidx, N_CONES),
                   axis=(0, 1), keepdims=True)
    out_ref[...] = cone[0, 0]


def _tc_cone_ids(pred_rotmats, grid_rotmats):
    b = pred_rotmats.shape[0]
    predT = pred_rotmats.transpose(1, 2, 0)
    gridT = grid_rotmats.transpose(1, 2, 0)
    return pl.pallas_call(
        _cone_kernel,
        grid=(b // BLOCK_B,),
        in_specs=[
            pl.BlockSpec((3, 3, BLOCK_B), lambda i: (0, 0, i)),
            pl.BlockSpec((3, 3, N_SO3), lambda i: (0, 0, 0)),
        ],
        out_specs=pl.BlockSpec((BLOCK_B,), lambda i: (i,)),
        out_shape=jax.ShapeDtypeStruct((b,), jnp.int32),
        compiler_params=pltpu.CompilerParams(
            dimension_semantics=("parallel",)),
    )(predT, gridT)


def _sc_normalize(cone_ids, scores, medians, mads):
    b = scores.shape[0]
    info = plsc.get_sparse_core_info()
    nw = info.num_cores * info.num_subcores
    bpw = b // nw
    mesh = plsc.VectorSubcoreMesh(core_axis_name="c", subcore_axis_name="s")

    @functools.partial(
        pl.kernel, mesh=mesh,
        out_type=jax.ShapeDtypeStruct((b,), jnp.float32),
        scratch_types=[
            pltpu.VMEM((bpw,), jnp.int32),
            pltpu.VMEM((bpw,), jnp.float32),
            pltpu.VMEM((bpw,), jnp.float32),
            pltpu.VMEM((bpw,), jnp.float32),
            pltpu.VMEM((bpw,), jnp.float32),
            pltpu.SemaphoreType.DMA,
        ],
    )
    def k(cone_hbm, scores_hbm, med_hbm, mad_hbm, out_hbm,
          idx_v, sc_v, med_g, mad_g, res_v, sem):
        wid = lax.axis_index("s") * info.num_cores + lax.axis_index("c")
        base = wid * bpw
        pltpu.sync_copy(cone_hbm.at[pl.ds(base, bpw)], idx_v)
        pltpu.sync_copy(scores_hbm.at[pl.ds(base, bpw)], sc_v)
        pltpu.async_copy(med_hbm.at[idx_v], med_g, sem).wait()
        pltpu.async_copy(mad_hbm.at[idx_v], mad_g, sem).wait()
        for j in range(bpw // 16):
            sl = pl.ds(j * 16, 16)
            res_v[sl] = (sc_v[sl] - med_g[sl]) / mad_g[sl]
        pltpu.sync_copy(res_v, out_hbm.at[pl.ds(base, bpw)])

    return k(cone_ids, scores, medians, mads)


@jax.jit
def kernel(pred_rotmats, scores, grid_rotmats, medians, mads):
    cone_ids = _tc_cone_ids(pred_rotmats, grid_rotmats)
    return _sc_normalize(cone_ids, scores, medians, mads)


# speedup vs baseline: 3.8640x; 3.8640x over previous
import jax
import jax.numpy as jnp
from jax.experimental import pallas as pl
from jax.experimental.pallas import tpu as pltpu

N_PSI = 24
N_CONES = 192
N_SO3 = N_CONES * N_PSI
BLOCK_B = 2048


def _fused_kernel(pred_ref, grid_ref, scores_ref, med_ref, mad_ref, out_ref):
    bb = pred_ref.shape[2]
    p9 = jnp.concatenate([pred_ref[0], pred_ref[1], pred_ref[2]], axis=0)
    g9 = jnp.concatenate([grid_ref[0], grid_ref[1], grid_ref[2]], axis=0)
    simT = jax.lax.dot_general(
        g9, p9, (((0,), (0,)), ((), ())),
        preferred_element_type=jnp.float32)
    m8 = jnp.max(simT.reshape(N_CONES, 3, 8, bb), axis=1)
    gmax = jnp.max(m8, axis=(0, 1), keepdims=True)
    cidx = jax.lax.broadcasted_iota(jnp.int32, (N_CONES, 1, 1), 0)
    cone = jnp.min(jnp.where(m8 == gmax, cidx, N_CONES),
                   axis=(0, 1), keepdims=True)
    onehotT = (cone[0] == jax.lax.broadcasted_iota(
        jnp.int32, (N_CONES, 1), 0)).astype(jnp.float32)
    stats = jnp.concatenate([med_ref[...].reshape(1, N_CONES),
                             mad_ref[...].reshape(1, N_CONES)],
                            axis=0)
    st = jnp.dot(stats, onehotT, preferred_element_type=jnp.float32)
    res = (scores_ref[...].reshape(1, bb) - st[0:1, :]) / st[1:2, :]
    out_ref[...] = res.reshape(bb)


@jax.jit
def kernel(pred_rotmats, scores, grid_rotmats, medians, mads):
    b = pred_rotmats.shape[0]
    predT = pred_rotmats.transpose(1, 2, 0)
    gridT = grid_rotmats.transpose(1, 2, 0)

    out = pl.pallas_call(
        _fused_kernel,
        grid=(b // BLOCK_B,),
        in_specs=[
            pl.BlockSpec((3, 3, BLOCK_B), lambda i: (0, 0, i)),
            pl.BlockSpec((3, 3, N_SO3), lambda i: (0, 0, 0)),
            pl.BlockSpec((BLOCK_B,), lambda i: (i,)),
            pl.BlockSpec((N_CONES,), lambda i: (0,)),
            pl.BlockSpec((N_CONES,), lambda i: (0,)),
        ],
        out_specs=pl.BlockSpec((BLOCK_B,), lambda i: (i,)),
        out_shape=jax.ShapeDtypeStruct((b,), jnp.float32),
        compiler_params=pltpu.CompilerParams(
            dimension_semantics=("parallel",)),
    )(predT, gridT, scores, medians, mads)
    return out
